# Initial kernel scaffold; baseline (speedup 1.0000x reference)
import functools

import jax
import jax.numpy as jnp
from jax import lax
from jax.experimental import pallas as pl
from jax.experimental.pallas import tpu as pltpu
from jax.experimental.pallas import tpu_sc as plsc

_RADIUS = 0.25
_K = 32
_N = 8192
_NB = 8
_L = 16


def _rev(v):
    return jnp.flip(v, axis=0)


def _merge(a0d, a0i, a1d, a1i, bd, bi):
    rbd, rbi = _rev(bd), _rev(bi)
    keep_a = a1d <= rbd
    md = jnp.where(keep_a, a1d, rbd)
    mi = jnp.where(keep_a, a1i, rbi)
    md, mi = plsc.sort_key_val(md, mi)
    rmd, rmi = _rev(md), _rev(mi)
    lo_is_a = a0d <= rmd
    sd = jnp.where(lo_is_a, a0d, rmd)
    si = jnp.where(lo_is_a, a0i, rmi)
    td = jnp.where(lo_is_a, rmd, a0d)
    ti = jnp.where(lo_is_a, rmi, a0i)
    n0d, n0i = plsc.sort_key_val(sd, si)
    n1d, n1i = plsc.sort_key_val(td, ti)
    return n0d, n0i, n1d, n1i


def _sc_body(xx_h, xy_h, xz_h, bat_h, src_h, tgt_h,
             xx, xy, xz, x2, bat, bnd, srcb, tgtb):
    info = plsc.get_sparse_core_info()
    nc = info.num_cores
    wid = lax.axis_index("s") * nc + lax.axis_index("c")
    nw = nc * info.num_subcores
    rows = _N // nw
    r0 = wid * rows

    pltpu.sync_copy(xx_h, xx)
    pltpu.sync_copy(xy_h, xy)
    pltpu.sync_copy(xz_h, xz)
    pltpu.sync_copy(bat_h, bat)

    def _x2_step(c, _):
        s = pl.ds(c * _L, _L)
        x2[s] = (xx[s] * xx[s] + xy[s] * xy[s]) + xz[s] * xz[s]
        return 0
    lax.fori_loop(0, _N // _L, _x2_step, 0)

    def _bisect(v, _):
        def _step(_, lh):
            l, h = lh
            mid = (l + h) // 2
            below = bat[mid] < v
            return (jnp.where(below, mid + 1, l), jnp.where(below, h, mid))
        l, _h = lax.fori_loop(0, 14, _step, (0, _N))
        bnd[v] = l
        return 0
    lax.fori_loop(0, _NB + 1, _bisect, 0)

    r2 = jnp.float32(_RADIUS * _RADIUS)
    inf16 = jnp.full((_L,), jnp.inf, jnp.float32)
    neg16 = jnp.full((_L,), -1, jnp.int32)
    iota = lax.broadcasted_iota(jnp.int32, (_L,), 0)

    def _row(r, _):
        i = r0 + r
        bi = bat[i]
        lo = bnd[bi]
        hi = bnd[bi + 1]
        xi0, xi1, xi2 = xx[i], xy[i], xz[i]
        x2i = x2[i]
        c0 = lo // _L
        c1 = (hi + _L - 1) // _L

        def _chunk(c, carry):
            a0d, a0i, a1d, a1i, thr = carry
            base = c * _L
            s = pl.ds(base, _L)
            jidx = base + iota
            dot = (xi0 * xx[s] + xi1 * xy[s]) + xi2 * xz[s]
            d2 = jnp.maximum((x2i + x2[s]) - 2.0 * dot, 0.0)
            valid = ((jidx >= lo) & (jidx < hi) & (jidx != i)
                     & (d2 <= r2))
            dk = jnp.where(valid, d2, jnp.inf)

            def _do(args):
                a0d, a0i, a1d, a1i, dk, jidx = args
                sd, si = plsc.sort_key_val(dk, jidx)
                n0d, n0i, n1d, n1i = _merge(a0d, a0i, a1d, a1i, sd, si)
                return n0d, n0i, n1d, n1i, jnp.max(n1d)

            def _skip(args):
                a0d, a0i, a1d, a1i, _dk, _j = args
                return a0d, a0i, a1d, a1i, thr

            return lax.cond(jnp.min(dk) < thr, _do, _skip,
                            (a0d, a0i, a1d, a1i, dk, jidx))

        init = (inf16, neg16, inf16, neg16, jnp.float32(jnp.inf))
        a0d, a0i, a1d, a1i, _t = lax.fori_loop(c0, c1, _chunk, init)

        o = pl.ds(r * _K, _L)
        o2 = pl.ds(r * _K + _L, _L)
        srcb[o] = a0i
        srcb[o2] = a1i
        tgtb[o] = jnp.where(a0d < jnp.inf, i, -1)
        tgtb[o2] = jnp.where(a1d < jnp.inf, i, -1)
        return 0

    lax.fori_loop(0, rows, _row, 0)

    out_s = pl.ds(r0 * _K, rows * _K)
    pltpu.sync_copy(srcb, src_h.at[out_s])
    pltpu.sync_copy(tgtb, tgt_h.at[out_s])


@jax.jit
def kernel(x, batch):
    n = x.shape[0]
    nk = n * _K
    rows = n // 32
    mesh = plsc.VectorSubcoreMesh(core_axis_name="c", subcore_axis_name="s")
    call = functools.partial(
        pl.kernel,
        mesh=mesh,
        out_type=[
            jax.ShapeDtypeStruct((nk,), jnp.int32),
            jax.ShapeDtypeStruct((nk,), jnp.int32),
        ],
        scratch_types=[
            pltpu.VMEM((n,), jnp.float32),
            pltpu.VMEM((n,), jnp.float32),
            pltpu.VMEM((n,), jnp.float32),
            pltpu.VMEM((n,), jnp.float32),
            pltpu.VMEM((n,), jnp.int32),
            pltpu.SMEM((16,), jnp.int32),
            pltpu.VMEM((rows * _K,), jnp.int32),
            pltpu.VMEM((rows * _K,), jnp.int32),
        ],
    )(_sc_body)
    src, tgt = call(
        x[:, 0].astype(jnp.float32),
        x[:, 1].astype(jnp.float32),
        x[:, 2].astype(jnp.float32),
        batch.astype(jnp.int32),
    )
    return jnp.stack([src, tgt], axis=0)


# speedup vs baseline: 21.9374x; 21.9374x over previous
import functools

import jax
import jax.numpy as jnp
from jax import lax
from jax.experimental import pallas as pl
from jax.experimental.pallas import tpu as pltpu
from jax.experimental.pallas import tpu_sc as plsc

_RADIUS = 0.25
_K = 32
_N = 8192
_NB = 8
_L = 16


def _rev(v):
    return jnp.flip(v, axis=0)


def _merge(a0d, a0i, a1d, a1i, bd, bi):
    rbd, rbi = _rev(bd), _rev(bi)
    keep_a = a1d <= rbd
    md = jnp.where(keep_a, a1d, rbd)
    mi = jnp.where(keep_a, a1i, rbi)
    md, mi = plsc.sort_key_val(md, mi)
    rmd, rmi = _rev(md), _rev(mi)
    lo_is_a = a0d <= rmd
    sd = jnp.where(lo_is_a, a0d, rmd)
    si = jnp.where(lo_is_a, a0i, rmi)
    td = jnp.where(lo_is_a, rmd, a0d)
    ti = jnp.where(lo_is_a, rmi, a0i)
    n0d, n0i = plsc.sort_key_val(sd, si)
    n1d, n1i = plsc.sort_key_val(td, ti)
    return n0d, n0i, n1d, n1i


def _vext(ref, idx):
    return ref[pl.ds(idx, _L)][0]


def _sc_body(xx_h, xy_h, xz_h, bat_h, src_h, tgt_h,
             xx, xy, xz, x2, bat, bnd, srcb, tgtb):
    info = plsc.get_sparse_core_info()
    nc = info.num_cores
    wid = lax.axis_index("s") * nc + lax.axis_index("c")
    nw = nc * info.num_subcores
    rows = _N // nw
    r0 = wid * rows

    pltpu.sync_copy(xx_h, xx.at[pl.ds(0, _N)])
    pltpu.sync_copy(xy_h, xy.at[pl.ds(0, _N)])
    pltpu.sync_copy(xz_h, xz.at[pl.ds(0, _N)])
    pltpu.sync_copy(bat_h, bat.at[pl.ds(0, _N)])

    def _rne(v):
        u = lax.bitcast_convert_type(v, jnp.int32)
        r = ((u >> 16) & 1) + 0x7FFF
        return lax.bitcast_convert_type((u + r) & jnp.int32(-65536),
                                        jnp.float32)

    def _x2_step(c, _):
        s = pl.ds(c * _L, _L)
        v0, v1, v2 = xx[s], xy[s], xz[s]
        x2[s] = (v0 * v0 + v1 * v1) + v2 * v2
        xx[s] = _rne(v0)
        xy[s] = _rne(v1)
        xz[s] = _rne(v2)
        return 0
    lax.fori_loop(0, _N // _L, _x2_step, 0)

    r2 = jnp.float32(_RADIUS * _RADIUS)
    inf16 = jnp.full((_L,), jnp.inf, jnp.float32)
    neg16 = jnp.full((_L,), -1, jnp.int32)
    iota = lax.broadcasted_iota(jnp.int32, (_L,), 0)

    def _bisect(v, _):
        def _step(_, lh):
            l, h = lh
            mid = (l + h) // 2
            below = _vext(bat, mid) < v
            return (jnp.where(below, mid + 1, l), jnp.where(below, h, mid))
        l, _h = lax.fori_loop(0, 14, _step, (0, _N))
        bnd[v] = l
        return 0
    lax.fori_loop(0, _NB + 1, _bisect, 0)

    def _row(r, _):
        i = r0 + r
        bi = _vext(bat, i)
        lo = bnd[bi]
        hi = bnd[bi + 1]
        xi0 = _vext(xx, i)
        xi1 = _vext(xy, i)
        xi2 = _vext(xz, i)
        x2i = _vext(x2, i)
        c0 = lo // _L
        c1 = (hi + _L - 1) // _L

        def _chunk(c, carry):
            a0d, a0i, a1d, a1i = carry
            base = c * _L
            s = pl.ds(base, _L)
            jidx = base + iota
            dot = (xi0 * xx[s] + xi1 * xy[s]) + xi2 * xz[s]
            d2 = jnp.maximum((x2i + x2[s]) - 2.0 * dot, 0.0)
            valid = ((jidx >= lo) & (jidx < hi) & (jidx != i)
                     & (d2 <= r2))
            dk = jnp.where(valid, d2, jnp.inf)
            sd, si = plsc.sort_key_val(dk, jidx)

            def _do(args):
                a0d, a0i, a1d, a1i, sd, si = args
                return _merge(a0d, a0i, a1d, a1i, sd, si)

            def _skip(args):
                a0d, a0i, a1d, a1i, _sd, _si = args
                return a0d, a0i, a1d, a1i

            return lax.cond(sd[0] < a1d[_L - 1], _do, _skip,
                            (a0d, a0i, a1d, a1i, sd, si))

        init = (inf16, neg16, inf16, neg16)
        a0d, a0i, a1d, a1i = lax.fori_loop(c0, c1, _chunk, init)

        o = pl.ds(r * _K, _L)
        o2 = pl.ds(r * _K + _L, _L)
        srcb[o] = a0i
        srcb[o2] = a1i
        tgtb[o] = jnp.where(a0d < jnp.inf, i, -1)
        tgtb[o2] = jnp.where(a1d < jnp.inf, i, -1)
        return 0

    lax.fori_loop(0, rows, _row, 0)

    out_s = pl.ds(r0 * _K, rows * _K)
    pltpu.sync_copy(srcb, src_h.at[out_s])
    pltpu.sync_copy(tgtb, tgt_h.at[out_s])


@jax.jit
def kernel(x, batch):
    n = x.shape[0]
    nk = n * _K
    rows = n // 32
    mesh = plsc.VectorSubcoreMesh(core_axis_name="c", subcore_axis_name="s")
    call = functools.partial(
        pl.kernel,
        mesh=mesh,
        compiler_params=pltpu.CompilerParams(needs_layout_passes=False),
        out_type=[
            jax.ShapeDtypeStruct((nk,), jnp.int32),
            jax.ShapeDtypeStruct((nk,), jnp.int32),
        ],
        scratch_types=[
            pltpu.VMEM((n + _L,), jnp.float32),
            pltpu.VMEM((n + _L,), jnp.float32),
            pltpu.VMEM((n + _L,), jnp.float32),
            pltpu.VMEM((n + _L,), jnp.float32),
            pltpu.VMEM((n + _L,), jnp.int32),
            pltpu.SMEM((16,), jnp.int32),
            pltpu.VMEM((rows * _K,), jnp.int32),
            pltpu.VMEM((rows * _K,), jnp.int32),
        ],
    )(_sc_body)
    src, tgt = call(
        x[:, 0].astype(jnp.float32),
        x[:, 1].astype(jnp.float32),
        x[:, 2].astype(jnp.float32),
        batch.astype(jnp.int32),
    )
    return jnp.stack([src, tgt], axis=0)


# speedup vs baseline: 40.1757x; 1.8314x over previous
import functools

import jax
import jax.numpy as jnp
from jax import lax
from jax.experimental import pallas as pl
from jax.experimental.pallas import tpu as pltpu
from jax.experimental.pallas import tpu_sc as plsc

_RADIUS = 0.25
_K = 32
_N = 8192
_NB = 8
_L = 16


def _rev(v):
    return jnp.flip(v, axis=0)


def _merge(a0d, a0i, a1d, a1i, bd, bi):
    rbd, rbi = _rev(bd), _rev(bi)
    keep_a = a1d <= rbd
    md = jnp.where(keep_a, a1d, rbd)
    mi = jnp.where(keep_a, a1i, rbi)
    md, mi = plsc.sort_key_val(md, mi)
    rmd, rmi = _rev(md), _rev(mi)
    lo_is_a = a0d <= rmd
    sd = jnp.where(lo_is_a, a0d, rmd)
    si = jnp.where(lo_is_a, a0i, rmi)
    td = jnp.where(lo_is_a, rmd, a0d)
    ti = jnp.where(lo_is_a, rmi, a0i)
    n0d, n0i = plsc.sort_key_val(sd, si)
    n1d, n1i = plsc.sort_key_val(td, ti)
    return n0d, n0i, n1d, n1i


def _vext(ref, idx):
    return ref[pl.ds(idx, _L)][0]


def _sc_body(xx_h, xy_h, xz_h, bat_h, src_h, tgt_h,
             xx, xy, xz, x2, bat, bnd, srcb, tgtb, cand_d, cand_i):
    info = plsc.get_sparse_core_info()
    nc = info.num_cores
    wid = lax.axis_index("s") * nc + lax.axis_index("c")
    nw = nc * info.num_subcores
    rows = _N // nw
    r0 = wid * rows

    pltpu.sync_copy(xx_h, xx.at[pl.ds(0, _N)])
    pltpu.sync_copy(xy_h, xy.at[pl.ds(0, _N)])
    pltpu.sync_copy(xz_h, xz.at[pl.ds(0, _N)])
    pltpu.sync_copy(bat_h, bat.at[pl.ds(0, _N)])

    def _rne(v):
        u = lax.bitcast_convert_type(v, jnp.int32)
        r = ((u >> 16) & 1) + 0x7FFF
        return lax.bitcast_convert_type((u + r) & jnp.int32(-65536),
                                        jnp.float32)

    def _x2_step(c, _):
        s = pl.ds(c * _L, _L)
        v0, v1, v2 = xx[s], xy[s], xz[s]
        x2[s] = (v0 * v0 + v1 * v1) + v2 * v2
        xx[s] = _rne(v0)
        xy[s] = _rne(v1)
        xz[s] = _rne(v2)
        return 0
    lax.fori_loop(0, _N // _L, _x2_step, 0)

    r2 = jnp.float32(_RADIUS * _RADIUS)
    inf16 = jnp.full((_L,), jnp.inf, jnp.float32)
    neg16 = jnp.full((_L,), -1, jnp.int32)
    iota = lax.broadcasted_iota(jnp.int32, (_L,), 0)

    def _bisect(v, _):
        def _step(_, lh):
            l, h = lh
            mid = (l + h) // 2
            below = _vext(bat, mid) < v
            return (jnp.where(below, mid + 1, l), jnp.where(below, h, mid))
        l, _h = lax.fori_loop(0, 14, _step, (0, _N))
        bnd[v] = l
        return 0
    lax.fori_loop(0, _NB + 1, _bisect, 0)

    def _row(r, _):
        i = r0 + r
        bi = _vext(bat, i)
        lo = bnd[bi]
        hi = bnd[bi + 1]
        xi0 = _vext(xx, i)
        xi1 = _vext(xy, i)
        xi2 = _vext(xz, i)
        x2i = _vext(x2, i)
        c0 = lo // _L
        c1 = (hi + _L - 1) // _L

        def _scan(c, cur):
            base = c * _L
            s = pl.ds(base, _L)
            jidx = base + iota
            dot = (xi0 * xx[s] + xi1 * xy[s]) + xi2 * xz[s]
            d2 = jnp.maximum((x2i + x2[s]) - 2.0 * dot, 0.0)
            valid = ((jidx >= lo) & (jidx < hi) & (jidx != i)
                     & (d2 <= r2))
            plsc.store_compressed(cand_d.at[pl.ds(cur, _L)], d2, mask=valid)
            plsc.store_compressed(cand_i.at[pl.ds(cur, _L)], jidx, mask=valid)
            return cur + plsc.all_reduce_population_count(valid)[0]

        cur = lax.fori_loop(c0, c1, _scan, 0)
        cand_d[pl.ds(cur, _L)] = inf16

        def _p2(c, carry):
            a0d, a0i, a1d, a1i = carry
            s = pl.ds(c * _L, _L)
            sd, si = plsc.sort_key_val(cand_d[s], cand_i[s])

            def _do(args):
                a0d, a0i, a1d, a1i, sd, si = args
                return _merge(a0d, a0i, a1d, a1i, sd, si)

            def _skip(args):
                a0d, a0i, a1d, a1i, _sd, _si = args
                return a0d, a0i, a1d, a1i

            return lax.cond(sd[0] < a1d[_L - 1], _do, _skip,
                            (a0d, a0i, a1d, a1i, sd, si))

        init = (inf16, neg16, inf16, neg16)
        a0d, a0i, a1d, a1i = lax.fori_loop(0, (cur + _L - 1) // _L, _p2, init)

        o = pl.ds(r * _K, _L)
        o2 = pl.ds(r * _K + _L, _L)
        srcb[o] = a0i
        srcb[o2] = a1i
        tgtb[o] = jnp.where(a0d < jnp.inf, i, -1)
        tgtb[o2] = jnp.where(a1d < jnp.inf, i, -1)
        return 0

    lax.fori_loop(0, rows, _row, 0)

    out_s = pl.ds(r0 * _K, rows * _K)
    pltpu.sync_copy(srcb, src_h.at[out_s])
    pltpu.sync_copy(tgtb, tgt_h.at[out_s])


@jax.jit
def kernel(x, batch):
    n = x.shape[0]
    nk = n * _K
    rows = n // 32
    mesh = plsc.VectorSubcoreMesh(core_axis_name="c", subcore_axis_name="s")
    call = functools.partial(
        pl.kernel,
        mesh=mesh,
        compiler_params=pltpu.CompilerParams(needs_layout_passes=False),
        out_type=[
            jax.ShapeDtypeStruct((nk,), jnp.int32),
            jax.ShapeDtypeStruct((nk,), jnp.int32),
        ],
        scratch_types=[
            pltpu.VMEM((n + _L,), jnp.float32),
            pltpu.VMEM((n + _L,), jnp.float32),
            pltpu.VMEM((n + _L,), jnp.float32),
            pltpu.VMEM((n + _L,), jnp.float32),
            pltpu.VMEM((n + _L,), jnp.int32),
            pltpu.SMEM((16,), jnp.int32),
            pltpu.VMEM((rows * _K,), jnp.int32),
            pltpu.VMEM((rows * _K,), jnp.int32),
            pltpu.VMEM((n + 2 * _L,), jnp.float32),
            pltpu.VMEM((n + 2 * _L,), jnp.int32),
        ],
    )(_sc_body)
    src, tgt = call(
        x[:, 0].astype(jnp.float32),
        x[:, 1].astype(jnp.float32),
        x[:, 2].astype(jnp.float32),
        batch.astype(jnp.int32),
    )
    return jnp.stack([src, tgt], axis=0)
